# Initial kernel scaffold; baseline (speedup 1.0000x reference)
import jax
import jax.numpy as jnp
from jax.experimental import pallas as pl


def kernel(query, pos_items, c0, c1, c0_, c1_, cd0, cd1, indices, indptr, wkk):
    raise NotImplementedError("write your pallas kernel here")


# speedup vs baseline: 15.5398x; 15.5398x over previous
import functools

import numpy as np
import jax
import jax.numpy as jnp
from jax import lax
from jax.experimental import pallas as pl
from jax.experimental.pallas import tpu as pltpu
from jax.experimental.pallas import tpu_sc as plsc

_B, _D, _K, _L, _NUM_NEG, _N = 4096, 64, 64, 20, 200, 100000


def _np_rotl(x, r):
    return ((x << np.uint32(r)) | (x >> np.uint32(32 - r))).astype(np.uint32)


def _np_threefry(k0, k1, x0, x1):
    x0 = np.asarray(x0, np.uint32).copy()
    x1 = np.asarray(x1, np.uint32).copy()
    ks = [np.uint32(k0), np.uint32(k1),
          np.uint32(np.uint32(k0) ^ np.uint32(k1) ^ np.uint32(0x1BD11BDA))]
    rots = [(13, 15, 26, 6), (17, 29, 16, 24)]
    x0 = (x0 + ks[0]).astype(np.uint32)
    x1 = (x1 + ks[1]).astype(np.uint32)
    for i in range(5):
        for r in rots[i % 2]:
            x0 = (x0 + x1).astype(np.uint32)
            x1 = _np_rotl(x1, r)
            x1 = (x1 ^ x0).astype(np.uint32)
        x0 = (x0 + ks[(i + 1) % 3]).astype(np.uint32)
        x1 = (x1 + ks[(i + 2) % 3] + np.uint32(i + 1)).astype(np.uint32)
    return x0, x1


def _subkey(i):
    v0, v1 = _np_threefry(0, 1234, np.zeros(1, np.uint32), np.full(1, i, np.uint32))
    return np.int32(v0[0].astype(np.int32)), np.int32(v1[0].astype(np.int32))


_KA = _subkey(0)
_KB = _subkey(1)
_KC = _subkey(2)

_TINY = np.float32(np.finfo(np.float32).tiny)
_ONE_MINUS_TINY = np.float32(np.float32(1.0) - _TINY)


def _tf_bits(cnt, key):
    k0 = jnp.int32(key[0])
    k1 = jnp.int32(key[1])
    ks2 = jnp.int32(np.int32(
        np.uint32(key[0]) ^ np.uint32(key[1]) ^ np.uint32(0x1BD11BDA)))
    ks = (k0, k1, ks2)
    rots = ((13, 15, 26, 6), (17, 29, 16, 24))
    x0 = jnp.zeros_like(cnt) + k0
    x1 = cnt + k1
    for i in range(5):
        for r in rots[i % 2]:
            x0 = x0 + x1
            x1 = lax.shift_left(x1, np.int32(r)) | lax.shift_right_logical(
                x1, np.int32(32 - r))
            x1 = x1 ^ x0
        x0 = x0 + ks[(i + 1) % 3]
        x1 = x1 + ks[(i + 2) % 3] + np.int32(i + 1)
    return x0 ^ x1


def _unit_float(bits):
    fb = lax.shift_right_logical(bits, np.int32(9)) | np.int32(0x3F800000)
    return lax.bitcast_convert_type(fb, jnp.float32) - np.float32(1.0)


def _gumbel(bits):
    f = _unit_float(bits)
    u = jnp.maximum(_TINY, f * _ONE_MINUS_TINY + _TINY)
    return -jnp.log(-jnp.log(u))


def _tc_body(q0t_ref, q1t_ref, c0t_ref, c1t_ref, wkk_ref, wkkt_ref,
             c0p_ref, c1p_ref, k0pt_ref, k1pt_ref,
             k01t_ref, pt_ref, ut_ref, post_ref):
    bb = q0t_ref.shape[1]
    b0 = pl.program_id(0) * bb
    q0t = q0t_ref[...]
    q1t = q1t_ref[...]
    r0t = jnp.dot(c0t_ref[...], q0t, preferred_element_type=jnp.float32)
    r1t = jnp.dot(c1t_ref[...], q1t, preferred_element_type=jnp.float32)

    def smx(v):
        m = jnp.max(v, axis=0, keepdims=True)
        e = jnp.exp(v - m)
        return e / jnp.sum(e, axis=0, keepdims=True)

    r1st = smx(r1t)
    r0st = smx(r0t)
    s0t = jnp.dot(wkk_ref[...], r1st, preferred_element_type=jnp.float32) * r0st
    l0t = jnp.log(s0t + np.float32(1e-20))
    wkkt = wkkt_ref[...]
    kiota = lax.broadcasted_iota(jnp.int32, (_K, bb), 0)
    lane2 = lax.broadcasted_iota(jnp.int32, (8 * _K, bb), 1)
    sio2 = lax.broadcasted_iota(jnp.int32, (8 * _K, bb), 0)
    cbase = (b0 + lane2) * np.int32(_NUM_NEG * _K) + sio2

    def chunk(jc, carry):
        cnt = cbase + jc * np.int32(8 * _K)
        g0c = _gumbel(_tf_bits(cnt, _KA))
        g1c = _gumbel(_tf_bits(cnt, _KB))
        krows = []
        prows = []
        for jj in range(8):
            sl = slice(jj * _K, (jj + 1) * _K)
            sc0 = l0t + g0c[sl, :]
            m0 = jnp.max(sc0, axis=0, keepdims=True)
            k0 = jnp.min(jnp.where(sc0 == m0, kiota, _K), axis=0, keepdims=True)
            oh0 = (kiota == k0).astype(jnp.float32)
            p0 = jnp.sum(r0t * oh0, axis=0, keepdims=True)
            subt = jnp.dot(wkkt, oh0, preferred_element_type=jnp.float32)
            sc1 = jnp.log(subt * r1st + np.float32(1e-20)) + g1c[sl, :]
            m1 = jnp.max(sc1, axis=0, keepdims=True)
            k1 = jnp.min(jnp.where(sc1 == m1, kiota, _K), axis=0, keepdims=True)
            oh1 = (kiota == k1).astype(jnp.float32)
            p1 = jnp.sum(r1t * oh1, axis=0, keepdims=True)
            krows.append(k0 * _K + k1)
            prows.append(p0 + p1)
        k01t_ref[pl.ds(jc * 8, 8), :] = jnp.concatenate(krows, axis=0)
        pt_ref[pl.ds(jc * 8, 8), :] = jnp.concatenate(prows, axis=0)
        return carry

    lax.fori_loop(0, _NUM_NEG // 8, chunk, 0)

    laneu = lax.broadcasted_iota(jnp.int32, (_NUM_NEG, bb), 1)
    siou = lax.broadcasted_iota(jnp.int32, (_NUM_NEG, bb), 0)
    ut_ref[...] = _unit_float(_tf_bits((b0 + laneu) * np.int32(_NUM_NEG) + siou, _KC))

    ni = 72
    piota = lax.broadcasted_iota(jnp.int32, (ni, bb), 0)
    q0r = q0t.astype(jnp.bfloat16).astype(jnp.float32)
    q1r = q1t.astype(jnp.bfloat16).astype(jnp.float32)
    c0p = c0p_ref[...]
    c1p = c1p_ref[...]
    k0pt = k0pt_ref[...]
    k1pt = k1pt_ref[...]
    rows = []
    for l in range(_L):
        a0 = k0pt[l:l + 1, :]
        a1 = k1pt[l:l + 1, :]
        a0 = jnp.where(a0 < 0, a0 + (_K + 1), a0)
        a1 = jnp.where(a1 < 0, a1 + (_K + 1), a1)
        o0 = (piota == a0).astype(jnp.float32)
        o1 = (piota == a1).astype(jnp.float32)
        g0 = jnp.dot(c0p, o0, preferred_element_type=jnp.float32)
        g1 = jnp.dot(c1p, o1, preferred_element_type=jnp.float32)
        rows.append(jnp.sum(g0 * q0r, axis=0, keepdims=True)
                    + jnp.sum(g1 * q1r, axis=0, keepdims=True))
    post_ref[...] = jnp.concatenate(rows, axis=0)


def _run_tc(q0t, q1t, c0t, c1t, wkk, wkkt, c0p, c1p, k0pt, k1pt,
            bb=128, interpret=False):
    grid = _B // bb
    full = lambda shape: pl.BlockSpec(shape, lambda i: (0, 0))
    col = lambda rows: pl.BlockSpec((rows, bb), lambda i: (0, i))
    return pl.pallas_call(
        _tc_body,
        grid=(grid,),
        in_specs=[
            col(_D // 2), col(_D // 2),
            full((_K, _D // 2)), full((_K, _D // 2)),
            full((_K, _K)), full((_K, _K)),
            full((_D // 2, 72)), full((_D // 2, 72)),
            col(_L), col(_L),
        ],
        out_specs=[col(_NUM_NEG), col(_NUM_NEG), col(_NUM_NEG), col(_L)],
        out_shape=[
            jax.ShapeDtypeStruct((_NUM_NEG, _B), jnp.int32),
            jax.ShapeDtypeStruct((_NUM_NEG, _B), jnp.float32),
            jax.ShapeDtypeStruct((_NUM_NEG, _B), jnp.float32),
            jax.ShapeDtypeStruct((_L, _B), jnp.float32),
        ],
        compiler_params=pltpu.CompilerParams(
            dimension_semantics=("arbitrary",)),
        interpret=interpret,
    )(q0t, q1t, c0t, c1t, wkk, wkkt, c0p, c1p, k0pt, k1pt)


_NW = 32


def _sc_gather_pair(cd0, cd1, pos_flat):
    n = pos_flat.shape[0]
    per = n // _NW
    tpad = 100016
    cd0p = jnp.pad(cd0, (0, tpad - cd0.shape[0]))
    cd1p = jnp.pad(cd1, (0, tpad - cd1.shape[0]))
    mesh = plsc.VectorSubcoreMesh(core_axis_name="c", subcore_axis_name="s")

    @functools.partial(
        pl.kernel, mesh=mesh,
        compiler_params=pltpu.CompilerParams(needs_layout_passes=False),
        out_type=[jax.ShapeDtypeStruct((n,), jnp.int32)] * 2,
        scratch_types=[pltpu.VMEM((tpad,), jnp.int32),
                       pltpu.VMEM((per,), jnp.int32),
                       pltpu.VMEM((per,), jnp.int32)])
    def go(cd0_h, cd1_h, pos_h, o0_h, o1_h, tab_v, idx_v, out_v):
        wid = lax.axis_index("s") * 2 + lax.axis_index("c")
        base = wid * per
        pltpu.sync_copy(pos_h.at[pl.ds(base, per)], idx_v)
        for tab_h, o_h in ((cd0_h, o0_h), (cd1_h, o1_h)):
            pltpu.sync_copy(tab_h, tab_v)

            def step(i, c):
                ii = idx_v[pl.ds(i * 16, 16)]
                out_v[pl.ds(i * 16, 16)] = plsc.load_gather(tab_v, [ii])
                return c

            lax.fori_loop(0, per // 16, step, 0)
            pltpu.sync_copy(out_v, o_h.at[pl.ds(base, per)])

    return go(cd0p, cd1p, pos_flat)


def _sc_sample_items(k01, u, indptr, indices):
    m = k01.shape[0]
    per = m // _NW
    ch = 3200
    ptr_pad = 4104
    indptr_p = jnp.pad(indptr, (0, ptr_pad - indptr.shape[0]))
    mesh = plsc.VectorSubcoreMesh(core_axis_name="c", subcore_axis_name="s")

    @functools.partial(
        pl.kernel, mesh=mesh,
        compiler_params=pltpu.CompilerParams(needs_layout_passes=False),
        out_type=jax.ShapeDtypeStruct((m,), jnp.int32),
        scratch_types=[pltpu.VMEM((_N,), jnp.int32),
                       pltpu.VMEM((ptr_pad,), jnp.int32),
                       pltpu.VMEM((ch,), jnp.int32),
                       pltpu.VMEM((ch,), jnp.float32),
                       pltpu.VMEM((ch,), jnp.int32)])
    def go(k01_h, u_h, ptr_h, ind_h, out_h, ind_v, ptr_v, k_v, u_v, o_v):
        wid = lax.axis_index("s") * 2 + lax.axis_index("c")
        base = wid * per
        pltpu.sync_copy(ind_h, ind_v)
        pltpu.sync_copy(ptr_h, ptr_v)

        def chunk(ci, c):
            off = base + ci * ch
            pltpu.sync_copy(k01_h.at[pl.ds(off, ch)], k_v)
            pltpu.sync_copy(u_h.at[pl.ds(off, ch)], u_v)

            def step(i, cc):
                kk = k_v[pl.ds(i * 16, 16)]
                i0 = plsc.load_gather(ptr_v, [kk])
                i1 = plsc.load_gather(ptr_v, [kk + 1])
                cntf = (i1 - i0).astype(jnp.float32)
                item = (cntf * u_v[pl.ds(i * 16, 16)]).astype(jnp.int32)
                addr = jnp.minimum(i0 + item, _N - 1)
                o_v[pl.ds(i * 16, 16)] = plsc.load_gather(ind_v, [addr]) + 1
                return cc

            lax.fori_loop(0, ch // 16, step, 0)
            pltpu.sync_copy(o_v, out_h.at[pl.ds(off, ch)])
            return c

        lax.fori_loop(0, per // ch, chunk, 0)

    return go(k01, u, indptr_p, indices)


def kernel(query, pos_items, c0, c1, c0_, c1_, cd0, cd1, indices, indptr, wkk):
    k0p_flat, k1p_flat = _sc_gather_pair(cd0, cd1, pos_items.reshape(-1))
    k0pt = k0p_flat.reshape(_B, _L).T
    k1pt = k1p_flat.reshape(_B, _L).T
    qt = query.T
    q0t = qt[:_D // 2]
    q1t = qt[_D // 2:]
    c0p = jnp.pad(c0_, ((0, 0), (0, 72 - (_K + 1))))
    c1p = jnp.pad(c1_, ((0, 0), (0, 72 - (_K + 1))))
    k01t, pt, ut, post = _run_tc(q0t, q1t, c0.T, c1.T, wkk, wkk.T,
                                 c0p, c1p, k0pt, k1pt)
    neg_flat = _sc_sample_items(k01t.T.reshape(-1), ut.T.reshape(-1),
                                indptr, indices)
    return (post.T, neg_flat.reshape(_B, _NUM_NEG), pt.T)
